# baseline (speedup 1.0000x reference)
import jax
import jax.numpy as jnp
from jax.experimental import pallas as pl
from jax.experimental.pallas import tpu as pltpu

_BS, _P, _C = 4, 100000, 16
_HO, _WO = 256, 256
_AA = 2
_H, _W = _HO * _AA, _WO * _AA


def _pool_body(cc_ref, out_ref):
    cc = cc_ref[0]
    canvas = cc[:, :_C]
    conf = cc[:, _C]
    inv = 1.0 / jnp.maximum(conf, 1e-8)
    m = (conf > 0.0)
    num = canvas * (inv * m.astype(jnp.float32))[:, None]
    num = num.reshape(2, _WO, 2, _C).sum(axis=(0, 2))
    den = m.astype(jnp.float32).reshape(2, _WO, 2).sum(axis=(0, 2))
    out_ref[0, 0] = num / jnp.maximum(den, 1.0)[:, None]


def _pool(cc):
    return pl.pallas_call(
        _pool_body,
        grid=(_BS, _HO),
        in_specs=[
            pl.BlockSpec((1, 2 * _W, _C + 1), lambda b, i: (b, i, 0)),
        ],
        out_specs=pl.BlockSpec((1, 1, _WO, _C), lambda b, i: (b, i, 0, 0)),
        out_shape=jax.ShapeDtypeStruct((_BS, _HO, _WO, _C), jnp.float32),
    )(cc)


def kernel(xyz, data, fov, h, w):
    z = xyz[..., 2]
    near = 0.99 * jnp.min(z, axis=1)
    far = jnp.quantile(z * (z < 100000.0).astype(z.dtype), 0.95, axis=1)
    far = jnp.maximum(far, near * 2.0)

    x, y = xyz[..., 0], xyz[..., 1]
    t = jnp.tan(fov * 0.5)[:, None]
    aspect = w / h
    nx = x / (z * t * aspect)
    ny = y / (z * t)
    n = near[:, None]
    f = far[:, None]
    nz = (f + n) / (f - n) - 2.0 * f * n / ((f - n) * z)

    viz = (jnp.abs(nx) <= 1.0) & (jnp.abs(ny) <= 1.0) & (nz >= -1.0) & (nz <= 1.0)
    u = (nx + 1.0) * 0.5 * (_W - 1)
    v = (ny + 1.0) * 0.5 * (_H - 1)
    u0 = jnp.floor(u)
    v0 = jnp.floor(v)
    du = u - u0
    dv = v - v0
    wz = jnp.exp(-2.5 * (nz + 1.0)) * viz.astype(jnp.float32)
    vals = jnp.transpose(data, (0, 2, 1))
    boff = (jnp.arange(_BS) * _H * _W)[:, None]
    canvas = jnp.zeros((_BS * _H * _W, _C), jnp.float32)
    conf = jnp.zeros((_BS * _H * _W,), jnp.float32)
    corners = ((0, 0, (1 - du) * (1 - dv)), (1, 0, du * (1 - dv)),
               (0, 1, (1 - du) * dv), (1, 1, du * dv))
    for di, dj, wc in corners:
        uu = u0 + di
        vv = v0 + dj
        inb = (uu >= 0) & (uu <= _W - 1) & (vv >= 0) & (vv <= _H - 1)
        wgt = wc * wz * inb.astype(jnp.float32)
        ui = jnp.clip(uu, 0, _W - 1).astype(jnp.int32)
        vi = jnp.clip(vv, 0, _H - 1).astype(jnp.int32)
        idx = (boff + vi * _W + ui).reshape(-1)
        canvas = canvas.at[idx].add((vals * wgt[..., None]).reshape(-1, _C))
        conf = conf.at[idx].add(wgt.reshape(-1))

    cc = jnp.concatenate(
        [canvas.reshape(_BS, _H * _W, _C),
         conf.reshape(_BS, _H * _W, 1)], axis=-1)
    out = _pool(cc)
    return jnp.transpose(out, (0, 3, 1, 2))


# speedup vs baseline: 1.2872x; 1.2872x over previous
import jax
import jax.numpy as jnp
from jax import lax
from jax.experimental import pallas as pl
from jax.experimental.pallas import tpu as pltpu
from jax.experimental.pallas import tpu_sc as plsc

_BS, _P, _C = 4, 100000, 16
_HO, _WO = 256, 256
_AA = 2
_H, _W = _HO * _AA, _WO * _AA
_HW = _H * _W
_NS = 16
_PT = 6400
_CHUNK = 640
_NCH = _PT // _CHUNK
_EC = 4 * _CHUNK
_NSTREAM = _EC // 128
_NG = 3
_NR = 2
_RG = _HW // _NR
_RSTRIPE = _RG // _NS
_ZCH = 1024
_TRASH = _HW


def _splat_body(idx_hbm, wgt_hbm, vals_hbm, zeros_hbm, out_hbm, canvas_sh):
    pl.run_scoped(
        lambda idx_v, idx_loc, wgt_v, vals_v, rows_v, zeros_v: _splat_inner(
            idx_hbm, wgt_hbm, vals_hbm, zeros_hbm, out_hbm, canvas_sh,
            idx_v, idx_loc, wgt_v, vals_v, rows_v, zeros_v),
        pltpu.VMEM((_NSTREAM, 128), jnp.int32),
        pltpu.VMEM((_NSTREAM, 128), jnp.int32),
        pltpu.VMEM((_EC,), jnp.float32),
        pltpu.VMEM((_CHUNK, _C), jnp.float32),
        pltpu.VMEM((_EC, 8), jnp.float32),
        pltpu.VMEM((_ZCH, 8), jnp.float32),
    )


def _splat_inner(idx_hbm, wgt_hbm, vals_hbm, zeros_hbm, out_hbm, canvas_sh,
                 idx_v, idx_loc, wgt_v, vals_v, rows_v, zeros_v):
    c = lax.axis_index("c")
    s = lax.axis_index("s")

    pltpu.sync_copy(zeros_hbm, zeros_v)

    def per_pass(v, carry):
        b = 2 * c + v // (_NR * _NG)
        r = (v // _NG) % _NR
        g = v % _NG

        def per_zero(zi, cz):
            pltpu.sync_copy(
                zeros_v,
                canvas_sh.at[pl.ds(s * _RSTRIPE + zi * _ZCH, _ZCH)])
            return cz

        lax.fori_loop(0, _RSTRIPE // _ZCH, per_zero, 0)
        plsc.subcore_barrier()

        def per_chunk(cn, c2):
            pltpu.sync_copy(idx_hbm.at[b, s, pl.ds(cn * _NSTREAM, _NSTREAM)],
                            idx_v)
            pltpu.sync_copy(wgt_hbm.at[b, s, pl.ds(cn * _EC, _EC)], wgt_v)
            pltpu.sync_copy(vals_hbm.at[b, s, pl.ds(cn * _CHUNK, _CHUNK)],
                            vals_v)

            def per_rw(t, c3):
                lanes = lax.iota(jnp.int32, 16)
                jrow = jnp.full((16,), t // 8, jnp.int32)
                kcol = jnp.full((16,), 16 * (t % 8), jnp.int32) + lanes
                gi = plsc.load_gather(idx_v, [jrow, kcol])
                li = gi - jnp.full((16,), r * _RG, jnp.int32)
                rgf = jnp.full((16,), _RG, jnp.int32)
                ok = (li >= jnp.full((16,), 0, jnp.int32)) & (li < rgf)
                plsc.store_scatter(idx_loc, [jrow, kcol],
                                   jnp.where(ok, li, rgf))
                return c3

            lax.fori_loop(0, 8 * _NSTREAM, per_rw, 0)

            def per_ent(m, c3):
                lanes = lax.iota(jnp.int32, 16)
                pat_a = lanes % 8
                pat_b = lanes // 8
                row_i = jnp.full((16,), m // 2, jnp.int32)
                ent_i = jnp.full((16,), 2 * m, jnp.int32) + pat_b
                col = jnp.minimum(jnp.full((16,), 8 * g, jnp.int32) + pat_a,
                                  jnp.full((16,), _C - 1, jnp.int32))
                v8 = plsc.load_gather(vals_v, [row_i, col])
                wv = plsc.load_gather(wgt_v, [ent_i])
                ones = jnp.full((16,), 1.0, jnp.float32)
                zeros = jnp.full((16,), 0.0, jnp.float32)
                crow = jnp.where(pat_a == jnp.full((16,), 0, jnp.int32),
                                 ones, zeros)
                cmask = jnp.full((16,), g == _NG - 1)
                row = jnp.where(cmask, crow, v8) * wv
                plsc.store_scatter(rows_v, [ent_i, pat_a], row)
                return c3

            lax.fori_loop(0, _EC // 2, per_ent, 0)

            def per_stream(j, c3):
                pltpu.sync_copy(rows_v.at[pl.ds(j * 128, 128)],
                                canvas_sh.at[idx_loc.at[j]], add=True)
                return c3

            lax.fori_loop(0, _NSTREAM, per_stream, 0)
            return c2

        lax.fori_loop(0, _NCH, per_chunk, 0)
        plsc.subcore_barrier()
        pltpu.sync_copy(
            canvas_sh.at[pl.ds(s * _RSTRIPE, _RSTRIPE)],
            out_hbm.at[b, g, pl.ds(r * _RG + s * _RSTRIPE, _RSTRIPE)])
        return carry

    lax.fori_loop(0, 2 * _NR * _NG, per_pass, 0)


def _splat(idx_e, wgt_e, vals_p, zeros):
    return pl.kernel(
        _splat_body,
        out_type=jax.ShapeDtypeStruct((_BS, _NG, _HW, 8), jnp.float32),
        mesh=plsc.VectorSubcoreMesh(core_axis_name="c", subcore_axis_name="s"),
        scratch_types=[
            pltpu.VMEM_SHARED((_RG + 128, 8), jnp.float32),
        ],
        compiler_params=pltpu.CompilerParams(use_tc_tiling_on_sc=False,
                                             needs_layout_passes=False),
    )(idx_e, wgt_e, vals_p, zeros)


def _pool_body(cc_ref, out_ref):
    cc = cc_ref[0]
    conf = cc[_NG - 1, :, 0]
    mf = (conf > 0.0).astype(jnp.float32)
    scale = ((1.0 / jnp.maximum(conf, 1e-8)) * mf)[:, None]
    outs = []
    for g in range(_NG - 1):
        dm = cc[g] * scale
        outs.append(dm.reshape(2, _WO, 2, 8).sum(axis=(0, 2)))
    den = mf.reshape(2, _WO, 2).sum(axis=(0, 2))
    num = jnp.concatenate(outs, axis=-1)
    out_ref[0, 0] = num / jnp.maximum(den, 1.0)[:, None]


def _pool(canvas):
    return pl.pallas_call(
        _pool_body,
        grid=(_BS, _HO),
        in_specs=[
            pl.BlockSpec((1, _NG, 2 * _W, 8), lambda b, i: (b, 0, i, 0)),
        ],
        out_specs=pl.BlockSpec((1, 1, _WO, _C), lambda b, i: (b, i, 0, 0)),
        out_shape=jax.ShapeDtypeStruct((_BS, _HO, _WO, _C), jnp.float32),
    )(canvas)


def kernel(xyz, data, fov, h, w):
    z = xyz[..., 2]
    near = 0.99 * jnp.min(z, axis=1)
    far = jnp.quantile(z * (z < 100000.0).astype(z.dtype), 0.95, axis=1)
    far = jnp.maximum(far, near * 2.0)

    x, y = xyz[..., 0], xyz[..., 1]
    t = jnp.tan(fov * 0.5)[:, None]
    aspect = w / h
    nx = x / (z * t * aspect)
    ny = y / (z * t)
    n = near[:, None]
    f = far[:, None]
    nz = (f + n) / (f - n) - 2.0 * f * n / ((f - n) * z)

    viz = (jnp.abs(nx) <= 1.0) & (jnp.abs(ny) <= 1.0) & (nz >= -1.0) & (nz <= 1.0)
    u = (nx + 1.0) * 0.5 * (_W - 1)
    v = (ny + 1.0) * 0.5 * (_H - 1)
    u0 = jnp.floor(u)
    v0 = jnp.floor(v)
    du = u - u0
    dv = v - v0
    wz = jnp.exp(-2.5 * (nz + 1.0)) * viz.astype(jnp.float32)

    idx_list, wgt_list = [], []
    corners = ((0, 0, (1 - du) * (1 - dv)), (1, 0, du * (1 - dv)),
               (0, 1, (1 - du) * dv), (1, 1, du * dv))
    for di, dj, wc in corners:
        uu = u0 + di
        vv = v0 + dj
        inb = (uu >= 0) & (uu <= _W - 1) & (vv >= 0) & (vv <= _H - 1)
        wgt = wc * wz * inb.astype(jnp.float32)
        ui = jnp.clip(uu, 0, _W - 1).astype(jnp.int32)
        vi = jnp.clip(vv, 0, _H - 1).astype(jnp.int32)
        idx = jnp.where(wgt > 0.0, vi * _W + ui, _TRASH)
        idx_list.append(idx)
        wgt_list.append(wgt)

    idx_e = jnp.stack(idx_list, axis=-1).reshape(_BS, _NS, _P // _NS, 4)
    wgt_e = jnp.stack(wgt_list, axis=-1).reshape(_BS, _NS, _P // _NS, 4)
    pad = _PT - _P // _NS
    idx_e = jnp.pad(idx_e, ((0, 0), (0, 0), (0, pad), (0, 0)),
                    constant_values=_TRASH)
    wgt_e = jnp.pad(wgt_e, ((0, 0), (0, 0), (0, pad), (0, 0)))
    idx_e = idx_e.reshape(_BS, _NS, _PT * 4 // 128, 128)
    wgt_e = wgt_e.reshape(_BS, _NS, _PT * 4)

    vals_p = jnp.transpose(data, (0, 2, 1)).reshape(_BS, _NS, _P // _NS, _C)
    vals_p = jnp.pad(vals_p, ((0, 0), (0, 0), (0, pad), (0, 0)))

    zeros = jnp.zeros((_ZCH, 8), jnp.float32)
    canvas = _splat(idx_e, wgt_e, vals_p, zeros)
    out = _pool(canvas)
    return jnp.transpose(out, (0, 3, 1, 2))


# speedup vs baseline: 1.3050x; 1.0138x over previous
import jax
import jax.numpy as jnp
from jax import lax
from jax.experimental import pallas as pl
from jax.experimental.pallas import tpu as pltpu
from jax.experimental.pallas import tpu_sc as plsc

_BS, _P, _C = 4, 100000, 16
_HO, _WO = 256, 256
_AA = 2
_H, _W = _HO * _AA, _WO * _AA
_HW = _H * _W
_NS = 16
_PT = 6400
_CHUNK = 640
_NCH = _PT // _CHUNK
_EC = 4 * _CHUNK
_NSTREAM = _EC // 128
_NG = 3
_NR = 2
_RG = _HW // _NR
_RSTRIPE = _RG // _NS
_ZCH = 1024
_TRASH = _HW


def _splat_body(idx_hbm, wgt_hbm, vals_hbm, zeros_hbm, out_hbm, canvas_sh):
    pl.run_scoped(
        lambda idx_v, idx_loc, wgt_v, vals_v, rows_v, zeros_v, dsem: \
            _splat_inner(
                idx_hbm, wgt_hbm, vals_hbm, zeros_hbm, out_hbm, canvas_sh,
                idx_v, idx_loc, wgt_v, vals_v, rows_v, zeros_v, dsem),
        pltpu.VMEM((_NSTREAM, 128), jnp.int32),
        pltpu.VMEM((_NSTREAM, 128), jnp.int32),
        pltpu.VMEM((_EC,), jnp.float32),
        pltpu.VMEM((_CHUNK, _C), jnp.float32),
        pltpu.VMEM((_EC, 8), jnp.float32),
        pltpu.VMEM((_ZCH, 8), jnp.float32),
        pltpu.SemaphoreType.DMA,
    )


def _splat_inner(idx_hbm, wgt_hbm, vals_hbm, zeros_hbm, out_hbm, canvas_sh,
                 idx_v, idx_loc, wgt_v, vals_v, rows_v, zeros_v, dsem):
    c = lax.axis_index("c")
    s = lax.axis_index("s")

    pltpu.sync_copy(zeros_hbm, zeros_v)

    def per_pass(v, carry):
        b = 2 * c + v // (_NR * _NG)
        r = (v // _NG) % _NR
        g = v % _NG

        def per_zero(zi, cz):
            pltpu.sync_copy(
                zeros_v,
                canvas_sh.at[pl.ds(s * _RSTRIPE + zi * _ZCH, _ZCH)])
            return cz

        lax.fori_loop(0, _RSTRIPE // _ZCH, per_zero, 0)
        plsc.subcore_barrier()

        def per_chunk(cn, c2):
            pltpu.sync_copy(idx_hbm.at[b, s, pl.ds(cn * _NSTREAM, _NSTREAM)],
                            idx_v)
            pltpu.sync_copy(wgt_hbm.at[b, s, pl.ds(cn * _EC, _EC)], wgt_v)
            pltpu.sync_copy(vals_hbm.at[b, s, pl.ds(cn * _CHUNK, _CHUNK)],
                            vals_v)

            def per_rw(t, c3):
                lanes = lax.iota(jnp.int32, 16)
                jrow = jnp.full((16,), t // 8, jnp.int32)
                kcol = jnp.full((16,), 16 * (t % 8), jnp.int32) + lanes
                gi = plsc.load_gather(idx_v, [jrow, kcol])
                li = gi - jnp.full((16,), r * _RG, jnp.int32)
                rgf = jnp.full((16,), _RG, jnp.int32)
                ok = (li >= jnp.full((16,), 0, jnp.int32)) & (li < rgf)
                plsc.store_scatter(idx_loc, [jrow, kcol],
                                   jnp.where(ok, li, rgf))
                return c3

            lax.fori_loop(0, 8 * _NSTREAM, per_rw, 0)

            def emit(m):
                lanes = lax.iota(jnp.int32, 16)
                pat_a = lanes % 8
                pat_b = lanes // 8
                row_i = jnp.full((16,), m // 2, jnp.int32)
                ent_i = jnp.full((16,), 2 * m, jnp.int32) + pat_b
                col = jnp.minimum(jnp.full((16,), 8 * g, jnp.int32) + pat_a,
                                  jnp.full((16,), _C - 1, jnp.int32))
                v8 = plsc.load_gather(vals_v, [row_i, col])
                wv = plsc.load_gather(wgt_v, [ent_i])
                ones = jnp.full((16,), 1.0, jnp.float32)
                zeros = jnp.full((16,), 0.0, jnp.float32)
                crow = jnp.where(pat_a == jnp.full((16,), 0, jnp.int32),
                                 ones, zeros)
                cmask = jnp.full((16,), g == _NG - 1)
                row = jnp.where(cmask, crow, v8) * wv
                plsc.store_scatter(rows_v, [ent_i, pat_a], row)

            def per_stream(j, c3):
                def per_ent2(mm, c4):
                    m0 = j * 64 + 2 * mm
                    emit(m0)
                    emit(m0 + 1)
                    return c4

                lax.fori_loop(0, 32, per_ent2, 0)
                pltpu.async_copy(rows_v.at[pl.ds(j * 128, 128)],
                                 canvas_sh.at[idx_loc.at[j]], dsem, add=True)
                return c3

            lax.fori_loop(0, _NSTREAM, per_stream, 0)

            def per_drain(j, c3):
                pltpu.make_async_copy(
                    rows_v.at[pl.ds(j * 128, 128)],
                    canvas_sh.at[idx_loc.at[j]], dsem).wait()
                return c3

            lax.fori_loop(0, _NSTREAM, per_drain, 0)
            return c2

        lax.fori_loop(0, _NCH, per_chunk, 0)
        plsc.subcore_barrier()
        pltpu.sync_copy(
            canvas_sh.at[pl.ds(s * _RSTRIPE, _RSTRIPE)],
            out_hbm.at[b, g, pl.ds(r * _RG + s * _RSTRIPE, _RSTRIPE)])
        return carry

    lax.fori_loop(0, 2 * _NR * _NG, per_pass, 0)


def _splat(idx_e, wgt_e, vals_p, zeros):
    return pl.kernel(
        _splat_body,
        out_type=jax.ShapeDtypeStruct((_BS, _NG, _HW, 8), jnp.float32),
        mesh=plsc.VectorSubcoreMesh(core_axis_name="c", subcore_axis_name="s"),
        scratch_types=[
            pltpu.VMEM_SHARED((_RG + 128, 8), jnp.float32),
        ],
        compiler_params=pltpu.CompilerParams(use_tc_tiling_on_sc=False,
                                             needs_layout_passes=False),
    )(idx_e, wgt_e, vals_p, zeros)


def _pool_body(cc_ref, out_ref):
    cc = cc_ref[0]
    conf = cc[_NG - 1, :, 0]
    mf = (conf > 0.0).astype(jnp.float32)
    scale = ((1.0 / jnp.maximum(conf, 1e-8)) * mf)[:, None]
    outs = []
    for g in range(_NG - 1):
        dm = cc[g] * scale
        outs.append(dm.reshape(2, _WO, 2, 8).sum(axis=(0, 2)))
    den = mf.reshape(2, _WO, 2).sum(axis=(0, 2))
    num = jnp.concatenate(outs, axis=-1)
    out_ref[0, 0] = num / jnp.maximum(den, 1.0)[:, None]


def _pool(canvas):
    return pl.pallas_call(
        _pool_body,
        grid=(_BS, _HO),
        in_specs=[
            pl.BlockSpec((1, _NG, 2 * _W, 8), lambda b, i: (b, 0, i, 0)),
        ],
        out_specs=pl.BlockSpec((1, 1, _WO, _C), lambda b, i: (b, i, 0, 0)),
        out_shape=jax.ShapeDtypeStruct((_BS, _HO, _WO, _C), jnp.float32),
    )(canvas)


def kernel(xyz, data, fov, h, w):
    z = xyz[..., 2]
    near = 0.99 * jnp.min(z, axis=1)
    far = jnp.quantile(z * (z < 100000.0).astype(z.dtype), 0.95, axis=1)
    far = jnp.maximum(far, near * 2.0)

    x, y = xyz[..., 0], xyz[..., 1]
    t = jnp.tan(fov * 0.5)[:, None]
    aspect = w / h
    nx = x / (z * t * aspect)
    ny = y / (z * t)
    n = near[:, None]
    f = far[:, None]
    nz = (f + n) / (f - n) - 2.0 * f * n / ((f - n) * z)

    viz = (jnp.abs(nx) <= 1.0) & (jnp.abs(ny) <= 1.0) & (nz >= -1.0) & (nz <= 1.0)
    u = (nx + 1.0) * 0.5 * (_W - 1)
    v = (ny + 1.0) * 0.5 * (_H - 1)
    u0 = jnp.floor(u)
    v0 = jnp.floor(v)
    du = u - u0
    dv = v - v0
    wz = jnp.exp(-2.5 * (nz + 1.0)) * viz.astype(jnp.float32)

    idx_list, wgt_list = [], []
    corners = ((0, 0, (1 - du) * (1 - dv)), (1, 0, du * (1 - dv)),
               (0, 1, (1 - du) * dv), (1, 1, du * dv))
    for di, dj, wc in corners:
        uu = u0 + di
        vv = v0 + dj
        inb = (uu >= 0) & (uu <= _W - 1) & (vv >= 0) & (vv <= _H - 1)
        wgt = wc * wz * inb.astype(jnp.float32)
        ui = jnp.clip(uu, 0, _W - 1).astype(jnp.int32)
        vi = jnp.clip(vv, 0, _H - 1).astype(jnp.int32)
        idx = jnp.where(wgt > 0.0, vi * _W + ui, _TRASH)
        idx_list.append(idx)
        wgt_list.append(wgt)

    idx_e = jnp.stack(idx_list, axis=-1).reshape(_BS, _NS, _P // _NS, 4)
    wgt_e = jnp.stack(wgt_list, axis=-1).reshape(_BS, _NS, _P // _NS, 4)
    pad = _PT - _P // _NS
    idx_e = jnp.pad(idx_e, ((0, 0), (0, 0), (0, pad), (0, 0)),
                    constant_values=_TRASH)
    wgt_e = jnp.pad(wgt_e, ((0, 0), (0, 0), (0, pad), (0, 0)))
    idx_e = idx_e.reshape(_BS, _NS, _PT * 4 // 128, 128)
    wgt_e = wgt_e.reshape(_BS, _NS, _PT * 4)

    vals_p = jnp.transpose(data, (0, 2, 1)).reshape(_BS, _NS, _P // _NS, _C)
    vals_p = jnp.pad(vals_p, ((0, 0), (0, 0), (0, pad), (0, 0)))

    zeros = jnp.zeros((_ZCH, 8), jnp.float32)
    canvas = _splat(idx_e, wgt_e, vals_p, zeros)
    out = _pool(canvas)
    return jnp.transpose(out, (0, 3, 1, 2))


# speedup vs baseline: 1.3051x; 1.0000x over previous
import jax
import jax.numpy as jnp
from jax import lax
from jax.experimental import pallas as pl
from jax.experimental.pallas import tpu as pltpu
from jax.experimental.pallas import tpu_sc as plsc

_BS, _P, _C = 4, 100000, 16
_HO, _WO = 256, 256
_AA = 2
_H, _W = _HO * _AA, _WO * _AA
_HW = _H * _W
_NS = 16
_PT = 6400
_CHUNK = 640
_NCH = _PT // _CHUNK
_EC = 4 * _CHUNK
_NSTREAM = _EC // 128
_NG = 3
_NR = 2
_RG = _HW // _NR
_RSTRIPE = _RG // _NS
_ZCH = 1024
_TRASH = _HW


def _splat_body(idx_hbm, wgt_hbm, vals_hbm, zeros_hbm, out_hbm, canvas_sh):
    pl.run_scoped(
        lambda idx_v, idx_loc, wgt_v, vals_v, rows_v, zeros_v, dsem: \
            _splat_inner(
                idx_hbm, wgt_hbm, vals_hbm, zeros_hbm, out_hbm, canvas_sh,
                idx_v, idx_loc, wgt_v, vals_v, rows_v, zeros_v, dsem),
        pltpu.VMEM((_NSTREAM, 128), jnp.int32),
        pltpu.VMEM((_NSTREAM, 128), jnp.int32),
        pltpu.VMEM((_EC,), jnp.float32),
        pltpu.VMEM((_CHUNK, _C), jnp.float32),
        pltpu.VMEM((_EC, 8), jnp.float32),
        pltpu.VMEM((_ZCH, 8), jnp.float32),
        pltpu.SemaphoreType.DMA,
    )


def _splat_inner(idx_hbm, wgt_hbm, vals_hbm, zeros_hbm, out_hbm, canvas_sh,
                 idx_v, idx_loc, wgt_v, vals_v, rows_v, zeros_v, dsem):
    c = lax.axis_index("c")
    s = lax.axis_index("s")

    pltpu.sync_copy(zeros_hbm, zeros_v)

    def per_pass(v, carry):
        b = 2 * c + v // (_NR * _NG)
        r = (v // _NG) % _NR
        g = v % _NG

        def per_zero(zi, cz):
            pltpu.sync_copy(
                zeros_v,
                canvas_sh.at[pl.ds(s * _RSTRIPE + zi * _ZCH, _ZCH)])
            return cz

        lax.fori_loop(0, _RSTRIPE // _ZCH, per_zero, 0)
        plsc.subcore_barrier()

        def per_chunk(cn, c2):
            pltpu.sync_copy(idx_hbm.at[b, s, pl.ds(cn * _NSTREAM, _NSTREAM)],
                            idx_v)
            pltpu.sync_copy(wgt_hbm.at[b, s, pl.ds(cn * _EC, _EC)], wgt_v)
            pltpu.sync_copy(vals_hbm.at[b, s, pl.ds(cn * _CHUNK, _CHUNK)],
                            vals_v)

            def per_rw(t, c3):
                lanes = lax.iota(jnp.int32, 16)
                jrow = jnp.full((16,), t // 8, jnp.int32)
                kcol = jnp.full((16,), 16 * (t % 8), jnp.int32) + lanes
                gi = plsc.load_gather(idx_v, [jrow, kcol])
                li = gi - jnp.full((16,), r * _RG, jnp.int32)
                rgf = jnp.full((16,), _RG, jnp.int32)
                ok = (li >= jnp.full((16,), 0, jnp.int32)) & (li < rgf)
                plsc.store_scatter(idx_loc, [jrow, kcol],
                                   jnp.where(ok, li, rgf))
                return c3

            lax.fori_loop(0, 8 * _NSTREAM, per_rw, 0)

            def emit(m):
                lanes = lax.iota(jnp.int32, 16)
                pat_a = lanes % 8
                pat_b = lanes // 8
                row_i = jnp.full((16,), m // 2, jnp.int32)
                ent_i = jnp.full((16,), 2 * m, jnp.int32) + pat_b
                col = jnp.minimum(jnp.full((16,), 8 * g, jnp.int32) + pat_a,
                                  jnp.full((16,), _C - 1, jnp.int32))
                v8 = plsc.load_gather(vals_v, [row_i, col])
                wv = plsc.load_gather(wgt_v, [ent_i])
                ones = jnp.full((16,), 1.0, jnp.float32)
                zeros = jnp.full((16,), 0.0, jnp.float32)
                crow = jnp.where(pat_a == jnp.full((16,), 0, jnp.int32),
                                 ones, zeros)
                cmask = jnp.full((16,), g == _NG - 1)
                row = jnp.where(cmask, crow, v8) * wv
                plsc.store_scatter(rows_v, [ent_i, pat_a], row)

            def per_stream(j, c3):
                def per_ent2(mm, c4):
                    m0 = j * 64 + 4 * mm
                    emit(m0)
                    emit(m0 + 1)
                    emit(m0 + 2)
                    emit(m0 + 3)
                    return c4

                lax.fori_loop(0, 16, per_ent2, 0)
                pltpu.sync_copy(rows_v.at[pl.ds(j * 128, 128)],
                                canvas_sh.at[idx_loc.at[j]], add=True)
                return c3

            lax.fori_loop(0, _NSTREAM, per_stream, 0)
            return c2

        lax.fori_loop(0, _NCH, per_chunk, 0)
        plsc.subcore_barrier()
        pltpu.sync_copy(
            canvas_sh.at[pl.ds(s * _RSTRIPE, _RSTRIPE)],
            out_hbm.at[b, g, pl.ds(r * _RG + s * _RSTRIPE, _RSTRIPE)])
        return carry

    lax.fori_loop(0, 2 * _NR * _NG, per_pass, 0)


def _splat(idx_e, wgt_e, vals_p, zeros):
    return pl.kernel(
        _splat_body,
        out_type=jax.ShapeDtypeStruct((_BS, _NG, _HW, 8), jnp.float32),
        mesh=plsc.VectorSubcoreMesh(core_axis_name="c", subcore_axis_name="s"),
        scratch_types=[
            pltpu.VMEM_SHARED((_RG + 128, 8), jnp.float32),
        ],
        compiler_params=pltpu.CompilerParams(use_tc_tiling_on_sc=False,
                                             needs_layout_passes=False),
    )(idx_e, wgt_e, vals_p, zeros)


def _pool_body(cc_ref, out_ref):
    cc = cc_ref[0]
    conf = cc[_NG - 1, :, 0]
    mf = (conf > 0.0).astype(jnp.float32)
    scale = ((1.0 / jnp.maximum(conf, 1e-8)) * mf)[:, None]
    outs = []
    for g in range(_NG - 1):
        dm = cc[g] * scale
        outs.append(dm.reshape(2, _WO, 2, 8).sum(axis=(0, 2)))
    den = mf.reshape(2, _WO, 2).sum(axis=(0, 2))
    num = jnp.concatenate(outs, axis=-1)
    out_ref[0, 0] = num / jnp.maximum(den, 1.0)[:, None]


def _pool(canvas):
    return pl.pallas_call(
        _pool_body,
        grid=(_BS, _HO),
        in_specs=[
            pl.BlockSpec((1, _NG, 2 * _W, 8), lambda b, i: (b, 0, i, 0)),
        ],
        out_specs=pl.BlockSpec((1, 1, _WO, _C), lambda b, i: (b, i, 0, 0)),
        out_shape=jax.ShapeDtypeStruct((_BS, _HO, _WO, _C), jnp.float32),
    )(canvas)


def kernel(xyz, data, fov, h, w):
    z = xyz[..., 2]
    near = 0.99 * jnp.min(z, axis=1)
    far = jnp.quantile(z * (z < 100000.0).astype(z.dtype), 0.95, axis=1)
    far = jnp.maximum(far, near * 2.0)

    x, y = xyz[..., 0], xyz[..., 1]
    t = jnp.tan(fov * 0.5)[:, None]
    aspect = w / h
    nx = x / (z * t * aspect)
    ny = y / (z * t)
    n = near[:, None]
    f = far[:, None]
    nz = (f + n) / (f - n) - 2.0 * f * n / ((f - n) * z)

    viz = (jnp.abs(nx) <= 1.0) & (jnp.abs(ny) <= 1.0) & (nz >= -1.0) & (nz <= 1.0)
    u = (nx + 1.0) * 0.5 * (_W - 1)
    v = (ny + 1.0) * 0.5 * (_H - 1)
    u0 = jnp.floor(u)
    v0 = jnp.floor(v)
    du = u - u0
    dv = v - v0
    wz = jnp.exp(-2.5 * (nz + 1.0)) * viz.astype(jnp.float32)

    idx_list, wgt_list = [], []
    corners = ((0, 0, (1 - du) * (1 - dv)), (1, 0, du * (1 - dv)),
               (0, 1, (1 - du) * dv), (1, 1, du * dv))
    for di, dj, wc in corners:
        uu = u0 + di
        vv = v0 + dj
        inb = (uu >= 0) & (uu <= _W - 1) & (vv >= 0) & (vv <= _H - 1)
        wgt = wc * wz * inb.astype(jnp.float32)
        ui = jnp.clip(uu, 0, _W - 1).astype(jnp.int32)
        vi = jnp.clip(vv, 0, _H - 1).astype(jnp.int32)
        idx = jnp.where(wgt > 0.0, vi * _W + ui, _TRASH)
        idx_list.append(idx)
        wgt_list.append(wgt)

    idx_e = jnp.stack(idx_list, axis=-1).reshape(_BS, _NS, _P // _NS, 4)
    wgt_e = jnp.stack(wgt_list, axis=-1).reshape(_BS, _NS, _P // _NS, 4)
    pad = _PT - _P // _NS
    idx_e = jnp.pad(idx_e, ((0, 0), (0, 0), (0, pad), (0, 0)),
                    constant_values=_TRASH)
    wgt_e = jnp.pad(wgt_e, ((0, 0), (0, 0), (0, pad), (0, 0)))
    idx_e = idx_e.reshape(_BS, _NS, _PT * 4 // 128, 128)
    wgt_e = wgt_e.reshape(_BS, _NS, _PT * 4)

    vals_p = jnp.transpose(data, (0, 2, 1)).reshape(_BS, _NS, _P // _NS, _C)
    vals_p = jnp.pad(vals_p, ((0, 0), (0, 0), (0, pad), (0, 0)))

    zeros = jnp.zeros((_ZCH, 8), jnp.float32)
    canvas = _splat(idx_e, wgt_e, vals_p, zeros)
    out = _pool(canvas)
    return jnp.transpose(out, (0, 3, 1, 2))


# speedup vs baseline: 1.3052x; 1.0001x over previous
import jax
import jax.numpy as jnp
from jax import lax
from jax.experimental import pallas as pl
from jax.experimental.pallas import tpu as pltpu
from jax.experimental.pallas import tpu_sc as plsc

_BS, _P, _C = 4, 100000, 16
_HO, _WO = 256, 256
_AA = 2
_H, _W = _HO * _AA, _WO * _AA
_HW = _H * _W
_NS = 16
_PT = 6400
_CHUNK = 640
_NCH = _PT // _CHUNK
_EC = 4 * _CHUNK
_NSTREAM = _EC // 128
_NG = 3
_NR = 2
_RG = _HW // _NR
_RSTRIPE = _RG // _NS
_ZCH = 1024
_TRASH = _HW


def _splat_body(idx_hbm, wgt_hbm, vals_hbm, zeros_hbm, out_hbm, canvas_sh):
    pl.run_scoped(
        lambda idx_v, idx_loc, wgt_v, vals_v, rows_v, zeros_v, dsem: \
            _splat_inner(
                idx_hbm, wgt_hbm, vals_hbm, zeros_hbm, out_hbm, canvas_sh,
                idx_v, idx_loc, wgt_v, vals_v, rows_v, zeros_v, dsem),
        pltpu.VMEM((_NSTREAM, 128), jnp.int32),
        pltpu.VMEM((_NSTREAM, 128), jnp.int32),
        pltpu.VMEM((_EC + 16,), jnp.float32),
        pltpu.VMEM((_CHUNK, _C), jnp.float32),
        pltpu.VMEM((_EC, 8), jnp.float32),
        pltpu.VMEM((_ZCH, 8), jnp.float32),
        pltpu.SemaphoreType.DMA,
    )


def _splat_inner(idx_hbm, wgt_hbm, vals_hbm, zeros_hbm, out_hbm, canvas_sh,
                 idx_v, idx_loc, wgt_v, vals_v, rows_v, zeros_v, dsem):
    c = lax.axis_index("c")
    s = lax.axis_index("s")

    pltpu.sync_copy(zeros_hbm, zeros_v)

    def per_pass(v, carry):
        b = 2 * c + v // (_NR * _NG)
        r = (v // _NG) % _NR
        g = v % _NG

        def per_zero(zi, cz):
            pltpu.sync_copy(
                zeros_v,
                canvas_sh.at[pl.ds(s * _RSTRIPE + zi * _ZCH, _ZCH)])
            return cz

        lax.fori_loop(0, _RSTRIPE // _ZCH, per_zero, 0)
        plsc.subcore_barrier()

        def per_chunk(cn, c2):
            pltpu.sync_copy(idx_hbm.at[b, s, pl.ds(cn * _NSTREAM, _NSTREAM)],
                            idx_v)
            pltpu.sync_copy(wgt_hbm.at[b, s, pl.ds(cn * _EC, _EC)],
                            wgt_v.at[pl.ds(0, _EC)])
            pltpu.sync_copy(vals_hbm.at[b, s, pl.ds(cn * _CHUNK, _CHUNK)],
                            vals_v)

            def per_rw(t, c3):
                lanes = lax.iota(jnp.int32, 16)
                jrow = jnp.full((16,), t // 8, jnp.int32)
                kcol = jnp.full((16,), 16 * (t % 8), jnp.int32) + lanes
                gi = plsc.load_gather(idx_v, [jrow, kcol])
                li = gi - jnp.full((16,), r * _RG, jnp.int32)
                rgf = jnp.full((16,), _RG, jnp.int32)
                ok = (li >= jnp.full((16,), 0, jnp.int32)) & (li < rgf)
                plsc.store_scatter(idx_loc, [jrow, kcol],
                                   jnp.where(ok, li, rgf))
                return c3

            lax.fori_loop(0, 8 * _NSTREAM, per_rw, 0)

            def emit(m):
                lanes = lax.iota(jnp.int32, 16)
                pat_a = lanes % 8
                pat_b = lanes // 8
                ent_i = jnp.full((16,), 2 * m, jnp.int32) + pat_b
                wpair = wgt_v[pl.ds(2 * m, 16)]
                w0 = jnp.full((16,), wpair[0], jnp.float32)
                w1 = jnp.full((16,), wpair[1], jnp.float32)
                wv = jnp.where(pat_b == jnp.full((16,), 0, jnp.int32),
                               w0, w1)
                col = jnp.minimum(jnp.full((16,), 8 * g, jnp.int32)
                                  + pat_a,
                                  jnp.full((16,), _C - 1, jnp.int32))
                row_i = jnp.full((16,), m // 2, jnp.int32)
                v8 = plsc.load_gather(vals_v, [row_i, col])
                ones = jnp.full((16,), 1.0, jnp.float32)
                zeros = jnp.full((16,), 0.0, jnp.float32)
                crow = jnp.where(pat_a == jnp.full((16,), 0, jnp.int32),
                                 ones, zeros)
                cmask = jnp.full((16,), g == _NG - 1)
                row = jnp.where(cmask, crow, v8) * wv
                plsc.store_scatter(rows_v, [ent_i, pat_a], row)

            def per_stream(j, c3):
                def per_ent2(mm, c4):
                    m0 = j * 64 + 4 * mm
                    emit(m0)
                    emit(m0 + 1)
                    emit(m0 + 2)
                    emit(m0 + 3)
                    return c4

                lax.fori_loop(0, 16, per_ent2, 0)
                pltpu.sync_copy(rows_v.at[pl.ds(j * 128, 128)],
                                canvas_sh.at[idx_loc.at[j]], add=True)
                return c3

            lax.fori_loop(0, _NSTREAM, per_stream, 0)
            return c2

        lax.fori_loop(0, _NCH, per_chunk, 0)
        plsc.subcore_barrier()
        pltpu.sync_copy(
            canvas_sh.at[pl.ds(s * _RSTRIPE, _RSTRIPE)],
            out_hbm.at[b, g, pl.ds(r * _RG + s * _RSTRIPE, _RSTRIPE)])
        return carry

    lax.fori_loop(0, 2 * _NR * _NG, per_pass, 0)


def _splat(idx_e, wgt_e, vals_p, zeros):
    return pl.kernel(
        _splat_body,
        out_type=jax.ShapeDtypeStruct((_BS, _NG, _HW, 8), jnp.float32),
        mesh=plsc.VectorSubcoreMesh(core_axis_name="c", subcore_axis_name="s"),
        scratch_types=[
            pltpu.VMEM_SHARED((_RG + 128, 8), jnp.float32),
        ],
        compiler_params=pltpu.CompilerParams(use_tc_tiling_on_sc=False,
                                             needs_layout_passes=False),
    )(idx_e, wgt_e, vals_p, zeros)


def _pool_body(cc_ref, out_ref):
    cc = cc_ref[0]
    conf = cc[_NG - 1, :, 0]
    mf = (conf > 0.0).astype(jnp.float32)
    scale = ((1.0 / jnp.maximum(conf, 1e-8)) * mf)[:, None]
    outs = []
    for g in range(_NG - 1):
        dm = cc[g] * scale
        outs.append(dm.reshape(2, _WO, 2, 8).sum(axis=(0, 2)))
    den = mf.reshape(2, _WO, 2).sum(axis=(0, 2))
    num = jnp.concatenate(outs, axis=-1)
    out_ref[0, 0] = num / jnp.maximum(den, 1.0)[:, None]


def _pool(canvas):
    return pl.pallas_call(
        _pool_body,
        grid=(_BS, _HO),
        in_specs=[
            pl.BlockSpec((1, _NG, 2 * _W, 8), lambda b, i: (b, 0, i, 0)),
        ],
        out_specs=pl.BlockSpec((1, 1, _WO, _C), lambda b, i: (b, i, 0, 0)),
        out_shape=jax.ShapeDtypeStruct((_BS, _HO, _WO, _C), jnp.float32),
    )(canvas)


def kernel(xyz, data, fov, h, w):
    z = xyz[..., 2]
    near = 0.99 * jnp.min(z, axis=1)
    far = jnp.quantile(z * (z < 100000.0).astype(z.dtype), 0.95, axis=1)
    far = jnp.maximum(far, near * 2.0)

    x, y = xyz[..., 0], xyz[..., 1]
    t = jnp.tan(fov * 0.5)[:, None]
    aspect = w / h
    nx = x / (z * t * aspect)
    ny = y / (z * t)
    n = near[:, None]
    f = far[:, None]
    nz = (f + n) / (f - n) - 2.0 * f * n / ((f - n) * z)

    viz = (jnp.abs(nx) <= 1.0) & (jnp.abs(ny) <= 1.0) & (nz >= -1.0) & (nz <= 1.0)
    u = (nx + 1.0) * 0.5 * (_W - 1)
    v = (ny + 1.0) * 0.5 * (_H - 1)
    u0 = jnp.floor(u)
    v0 = jnp.floor(v)
    du = u - u0
    dv = v - v0
    wz = jnp.exp(-2.5 * (nz + 1.0)) * viz.astype(jnp.float32)

    idx_list, wgt_list = [], []
    corners = ((0, 0, (1 - du) * (1 - dv)), (1, 0, du * (1 - dv)),
               (0, 1, (1 - du) * dv), (1, 1, du * dv))
    for di, dj, wc in corners:
        uu = u0 + di
        vv = v0 + dj
        inb = (uu >= 0) & (uu <= _W - 1) & (vv >= 0) & (vv <= _H - 1)
        wgt = wc * wz * inb.astype(jnp.float32)
        ui = jnp.clip(uu, 0, _W - 1).astype(jnp.int32)
        vi = jnp.clip(vv, 0, _H - 1).astype(jnp.int32)
        idx = jnp.where(wgt > 0.0, vi * _W + ui, _TRASH)
        idx_list.append(idx)
        wgt_list.append(wgt)

    idx_e = jnp.stack(idx_list, axis=-1).reshape(_BS, _NS, _P // _NS, 4)
    wgt_e = jnp.stack(wgt_list, axis=-1).reshape(_BS, _NS, _P // _NS, 4)
    pad = _PT - _P // _NS
    idx_e = jnp.pad(idx_e, ((0, 0), (0, 0), (0, pad), (0, 0)),
                    constant_values=_TRASH)
    wgt_e = jnp.pad(wgt_e, ((0, 0), (0, 0), (0, pad), (0, 0)))
    idx_e = idx_e.reshape(_BS, _NS, _PT * 4 // 128, 128)
    wgt_e = wgt_e.reshape(_BS, _NS, _PT * 4)

    vals_p = jnp.transpose(data, (0, 2, 1)).reshape(_BS, _NS, _P // _NS, _C)
    vals_p = jnp.pad(vals_p, ((0, 0), (0, 0), (0, pad), (0, 0)))

    zeros = jnp.zeros((_ZCH, 8), jnp.float32)
    canvas = _splat(idx_e, wgt_e, vals_p, zeros)
    out = _pool(canvas)
    return jnp.transpose(out, (0, 3, 1, 2))
